# baseline (speedup 1.0000x reference)
import jax
import jax.numpy as jnp
from jax.experimental import pallas as pl
from jax.experimental.pallas import tpu as pltpu

_HEAD = 8
_NCHUNK = 8


def _chunks(total):
    base = (total // (8 * _NCHUNK)) * 8
    sizes = [base] * (_NCHUNK - 1)
    sizes.append(total - base * (_NCHUNK - 1))
    return sizes


def _body(x_hbm, o_hbm, head_vmem, sem_bulk, sem_head):
    n = x_hbm.shape[0]
    sizes = _chunks(n - _HEAD)
    off = _HEAD
    copies = []
    for s in sizes:
        c = pltpu.make_async_copy(
            x_hbm.at[pl.ds(off, s)], o_hbm.at[pl.ds(off, s)], sem_bulk)
        c.start()
        copies.append(c)
        off += s

    head_in = pltpu.make_async_copy(x_hbm.at[pl.ds(0, _HEAD)], head_vmem, sem_head)
    head_in.start()
    head_in.wait()
    blk = head_vmem[...]
    r = jax.lax.broadcasted_iota(jnp.int32, blk.shape, 0)
    c = jax.lax.broadcasted_iota(jnp.int32, blk.shape, 1)
    col0 = c == 0
    blk = jnp.where((r == 0) & col0, jnp.float32(88.0), blk)
    blk = jnp.where((r == 1) & col0, jnp.float32(99.0), blk)
    head_vmem[...] = blk
    head_out = pltpu.make_async_copy(head_vmem, o_hbm.at[pl.ds(0, _HEAD)], sem_head)
    head_out.start()
    head_out.wait()

    for c in copies:
        c.wait()


def kernel(x):
    n, d = x.shape
    return pl.pallas_call(
        _body,
        in_specs=[pl.BlockSpec(memory_space=pl.ANY)],
        out_specs=pl.BlockSpec(memory_space=pl.ANY),
        out_shape=jax.ShapeDtypeStruct((n, d), x.dtype),
        scratch_shapes=[
            pltpu.VMEM((_HEAD, d), x.dtype),
            pltpu.SemaphoreType.DMA,
            pltpu.SemaphoreType.DMA,
        ],
    )(x)


# speedup vs baseline: 16.1374x; 16.1374x over previous
import jax
import jax.numpy as jnp
from jax import lax
from jax.experimental import pallas as pl
from jax.experimental.pallas import tpu as pltpu

_HEAD = 8
_CHUNK = 8192
_NBUF = 4
_SLOTS = 2 * _NBUF


def _body(x_hbm, o_hbm, head_vmem, bufs, tail_vmem,
          sem_in, sem_out, sem_head, sem_tail):
    n = x_hbm.shape[0]
    bulk = n - _HEAD
    k = bulk // _CHUNK
    rem = bulk - k * _CHUNK

    def in_copy(i, slot):
        return pltpu.make_async_copy(
            x_hbm.at[pl.ds(_HEAD + i * _CHUNK, _CHUNK)], bufs.at[slot],
            sem_in.at[slot])

    def out_copy(i, slot):
        return pltpu.make_async_copy(
            bufs.at[slot], o_hbm.at[pl.ds(_HEAD + i * _CHUNK, _CHUNK)],
            sem_out.at[slot])

    if rem:
        tail_in = pltpu.make_async_copy(
            x_hbm.at[pl.ds(_HEAD + k * _CHUNK, rem)], tail_vmem, sem_tail)
        tail_in.start()
    head_in = pltpu.make_async_copy(x_hbm.at[pl.ds(0, _HEAD)], head_vmem,
                                    sem_head)
    head_in.start()

    for j in range(min(_NBUF, k)):
        in_copy(j, j).start()

    def loop(i, _):
        j = i + _NBUF

        @pl.when(j < k)
        def _refill():
            slot_j = lax.rem(j, _SLOTS)

            @pl.when(j >= _SLOTS)
            def _free_slot():
                out_copy(j - _SLOTS, slot_j).wait()

            in_copy(j, slot_j).start()

        slot = lax.rem(i, _SLOTS)
        in_copy(i, slot).wait()
        out_copy(i, slot).start()
        return 0

    lax.fori_loop(0, k, loop, 0)

    head_in.wait()
    blk = head_vmem[...]
    r = lax.broadcasted_iota(jnp.int32, blk.shape, 0)
    c = lax.broadcasted_iota(jnp.int32, blk.shape, 1)
    col0 = c == 0
    blk = jnp.where((r == 0) & col0, jnp.float32(88.0), blk)
    blk = jnp.where((r == 1) & col0, jnp.float32(99.0), blk)
    head_vmem[...] = blk
    head_out = pltpu.make_async_copy(head_vmem, o_hbm.at[pl.ds(0, _HEAD)],
                                     sem_head)
    head_out.start()

    if rem:
        tail_in.wait()
        tail_out = pltpu.make_async_copy(
            tail_vmem, o_hbm.at[pl.ds(_HEAD + k * _CHUNK, rem)], sem_tail)
        tail_out.start()

    pend = min(k, _SLOTS)
    for t in range(pend):
        i = k - pend + t
        out_copy(i, i % _SLOTS).wait()
    head_out.wait()
    if rem:
        tail_out.wait()


def kernel(x):
    n, d = x.shape
    bulk = n - _HEAD
    rem = bulk - (bulk // _CHUNK) * _CHUNK
    return pl.pallas_call(
        _body,
        in_specs=[pl.BlockSpec(memory_space=pl.ANY)],
        out_specs=pl.BlockSpec(memory_space=pl.ANY),
        out_shape=jax.ShapeDtypeStruct((n, d), x.dtype),
        scratch_shapes=[
            pltpu.VMEM((_HEAD, d), x.dtype),
            pltpu.VMEM((_SLOTS, _CHUNK, d), x.dtype),
            pltpu.VMEM((max(rem, 1), d), x.dtype),
            pltpu.SemaphoreType.DMA((_SLOTS,)),
            pltpu.SemaphoreType.DMA((_SLOTS,)),
            pltpu.SemaphoreType.DMA,
            pltpu.SemaphoreType.DMA,
        ],
    )(x)
